# baseline (speedup 1.0000x reference)
import functools

import jax
import jax.numpy as jnp
from jax import lax
from jax.experimental import pallas as pl
from jax.experimental.pallas import tpu as pltpu
from jax.experimental.pallas import tpu_sc as plsc

B = 16384
D = 32
DC = 2 * D
NC = 2
NS = 16
NW = NC * NS
BPW = B // NW
CHUNK = 128
NCHUNK = BPW // CHUNK
L = 16

BLK = 2048
NB = B // (2 * BLK)


def _sc_gather_body(uid_hbm, mid_hbm, t_hbm, o_hbm,
                    uid_v, mid_v, urows, mrows, usem, msem, wsem):
  wid = lax.axis_index("s") * NC + lax.axis_index("c")
  base = wid * BPW
  pltpu.sync_copy(uid_hbm.at[pl.ds(wid * NCHUNK, NCHUNK)], uid_v)
  pltpu.sync_copy(mid_hbm.at[pl.ds(wid * NCHUNK, NCHUNK)], mid_v)
  for j in range(NCHUNK):
    for k in range(CHUNK // L):
      sl = pl.ds(k * L, L)
      u = uid_v[j, sl]
      uid_v[j, sl] = u + u
      m = mid_v[j, sl]
      mid_v[j, sl] = m + m + 1
  copies = []
  for j in range(NCHUNK):
    sl = pl.ds(j * CHUNK, CHUNK)
    copies.append(pltpu.async_copy(t_hbm.at[uid_v.at[j]], urows.at[sl],
                                   usem))
    copies.append(pltpu.async_copy(t_hbm.at[mid_v.at[j]], mrows.at[sl],
                                   msem))
  for c in copies:
    c.wait()
  wu = pltpu.async_copy(urows, o_hbm.at[pl.ds(base, BPW), pl.ds(0, DC)], wsem)
  wm = pltpu.async_copy(mrows, o_hbm.at[pl.ds(base, BPW), pl.ds(DC, DC)], wsem)
  wu.wait()
  wm.wait()


def _make_sc_gather():
  mesh = plsc.VectorSubcoreMesh(core_axis_name="c", subcore_axis_name="s",
                                num_cores=NC, num_subcores=NS)
  return pl.kernel(
      _sc_gather_body,
      out_type=jax.ShapeDtypeStruct((B, 2 * DC), jnp.float32),
      mesh=mesh,
      scratch_types=[
          pltpu.VMEM((NCHUNK, CHUNK), jnp.int32),
          pltpu.VMEM((NCHUNK, CHUNK), jnp.int32),
          pltpu.VMEM((BPW, DC), jnp.float32),
          pltpu.VMEM((BPW, DC), jnp.float32),
          pltpu.SemaphoreType.DMA,
          pltpu.SemaphoreType.DMA,
          pltpu.SemaphoreType.DMA,
      ],
      compiler_params=pltpu.CompilerParams(use_tc_tiling_on_sc=False),
  )


def _mlp_head(x, w0_ref, b0_ref, w1_ref, b1_ref, w2_ref, b2_ref,
              wn_ref, bn_ref):
  gmf = x[:, :D] * x[:, DC:DC + D]
  h = jnp.dot(x, w0_ref[...], preferred_element_type=jnp.float32)
  h = jnp.maximum(h + b0_ref[...], 0.0)
  h = jnp.maximum(
      jnp.dot(h, w1_ref[...], preferred_element_type=jnp.float32) + b1_ref[...],
      0.0)
  h = jnp.maximum(
      jnp.dot(h, w2_ref[...], preferred_element_type=jnp.float32) + b2_ref[...],
      0.0)
  wn = wn_ref[...]
  logit = jnp.sum(gmf * wn[:, :D], axis=1) + jnp.sum(h * wn[:, D:], axis=1)
  return (logit + bn_ref[0]).reshape(BLK // 128, 128)


def _tc_body(x1_ref, x2_ref, w0_ref, b0_ref, w1_ref, b1_ref,
             w2_ref, b2_ref, wn_ref, bn_ref, o1_ref, o2_ref):
  args = (w0_ref, b0_ref, w1_ref, b1_ref, w2_ref, b2_ref, wn_ref, bn_ref)
  o1_ref[...] = _mlp_head(x1_ref[...], *args)
  o2_ref[...] = _mlp_head(x2_ref[...], *args)


def _full(shape):
  return pl.BlockSpec(shape, lambda i: tuple(0 for _ in shape))


def _make_tc_dense():
  return pl.pallas_call(
      _tc_body,
      grid=(NB,),
      in_specs=[
          pl.BlockSpec((BLK, 2 * DC), lambda i: (i, 0)),
          pl.BlockSpec((BLK, 2 * DC), lambda i: (i + NB, 0)),
          _full((2 * DC, 128)),
          _full((1, 128)),
          _full((128, 64)),
          _full((1, 64)),
          _full((64, D)),
          _full((1, D)),
          _full((1, DC)),
          pl.BlockSpec(memory_space=pltpu.SMEM),
      ],
      out_specs=[
          pl.BlockSpec((BLK // 128, 128), lambda i: (i, 0)),
          pl.BlockSpec((BLK // 128, 128), lambda i: (i, 0)),
      ],
      out_shape=[
          jax.ShapeDtypeStruct((B // 256, 128), jnp.float32),
          jax.ShapeDtypeStruct((B // 256, 128), jnp.float32),
      ],
      compiler_params=pltpu.CompilerParams(
          dimension_semantics=("arbitrary",)),
  )


@jax.jit
def kernel(user_id, movie_title, user_gmf, movie_gmf, user_mlp, movie_mlp,
           W0, b0, W1, b1, W2, b2, Wn, bn):
  table = jnp.concatenate(
      [user_gmf, user_mlp, movie_gmf, movie_mlp], axis=1
  ).reshape(2 * user_gmf.shape[0], DC)
  uid = user_id.astype(jnp.int32).reshape(NW * NCHUNK, CHUNK)
  mid = movie_title.astype(jnp.int32).reshape(NW * NCHUNK, CHUNK)
  rows = _make_sc_gather()(uid, mid, table)
  x = rows.reshape(-1).reshape(B, 2 * DC)
  z = jnp.zeros((D, 128), jnp.float32)
  w0full = jnp.concatenate([z, W0[:D], z, W0[D:]], axis=0)
  o1, o2 = _make_tc_dense()(
      x, x, w0full, b0.reshape(1, 128), W1,
      b1.reshape(1, 64), W2, b2.reshape(1, D), Wn.reshape(1, DC), bn)
  return jnp.concatenate([o1, o2], axis=0).reshape(B)


# speedup vs baseline: 1.0704x; 1.0704x over previous
import functools

import jax
import jax.numpy as jnp
from jax import lax
from jax.experimental import pallas as pl
from jax.experimental.pallas import tpu as pltpu
from jax.experimental.pallas import tpu_sc as plsc

B = 16384
D = 32
DC = 2 * D
NC = 2
NS = 16
NW = NC * NS
BPW = B // NW
CHUNK = 128
NCHUNK = BPW // CHUNK
L = 16

BLK = 8192
NB = B // BLK


def _sc_gather_body(uid_hbm, mid_hbm, t_hbm, o_hbm,
                    uid_v, mid_v, urows, mrows, usem, msem, wsem):
  wid = lax.axis_index("s") * NC + lax.axis_index("c")
  base = wid * BPW
  cu = pltpu.async_copy(uid_hbm.at[pl.ds(wid * NCHUNK, NCHUNK)], uid_v, usem)
  cm = pltpu.async_copy(mid_hbm.at[pl.ds(wid * NCHUNK, NCHUNK)], mid_v, msem)
  copies = []
  cu.wait()
  for j in range(NCHUNK):
    for k in range(CHUNK // L):
      sl = pl.ds(k * L, L)
      u = uid_v[j, sl]
      uid_v[j, sl] = u + u
    copies.append(pltpu.async_copy(
        t_hbm.at[uid_v.at[j]], urows.at[pl.ds(j * CHUNK, CHUNK)], usem))
  cm.wait()
  for j in range(NCHUNK):
    for k in range(CHUNK // L):
      sl = pl.ds(k * L, L)
      m = mid_v[j, sl]
      mid_v[j, sl] = m + m + 1
    copies.append(pltpu.async_copy(
        t_hbm.at[mid_v.at[j]], mrows.at[pl.ds(j * CHUNK, CHUNK)], msem))
  for c in copies:
    c.wait()
  wu = pltpu.async_copy(urows, o_hbm.at[pl.ds(base, BPW), pl.ds(0, DC)], wsem)
  wm = pltpu.async_copy(mrows, o_hbm.at[pl.ds(base, BPW), pl.ds(DC, DC)], wsem)
  wu.wait()
  wm.wait()


def _make_sc_gather():
  mesh = plsc.VectorSubcoreMesh(core_axis_name="c", subcore_axis_name="s",
                                num_cores=NC, num_subcores=NS)
  return pl.kernel(
      _sc_gather_body,
      out_type=jax.ShapeDtypeStruct((B, 2 * DC), jnp.float32),
      mesh=mesh,
      scratch_types=[
          pltpu.VMEM((NCHUNK, CHUNK), jnp.int32),
          pltpu.VMEM((NCHUNK, CHUNK), jnp.int32),
          pltpu.VMEM((BPW, DC), jnp.float32),
          pltpu.VMEM((BPW, DC), jnp.float32),
          pltpu.SemaphoreType.DMA,
          pltpu.SemaphoreType.DMA,
          pltpu.SemaphoreType.DMA,
      ],
      compiler_params=pltpu.CompilerParams(use_tc_tiling_on_sc=False),
  )


def _mlp_head(x, w0_ref, b0_ref, w1_ref, b1_ref, w2_ref, b2_ref,
              wn_ref, bn_ref):
  gmf = x[:, :D] * x[:, DC:DC + D]
  h = jnp.dot(x, w0_ref[...], preferred_element_type=jnp.float32)
  h = jnp.maximum(h + b0_ref[...], 0.0)
  h = jnp.maximum(
      jnp.dot(h, w1_ref[...], preferred_element_type=jnp.float32) + b1_ref[...],
      0.0)
  h = jnp.maximum(
      jnp.dot(h, w2_ref[...], preferred_element_type=jnp.float32) + b2_ref[...],
      0.0)
  wn = wn_ref[...]
  logit = jnp.sum(gmf * wn[:, :D], axis=1) + jnp.sum(h * wn[:, D:], axis=1)
  return (logit + bn_ref[0]).reshape(BLK // 128, 128)


def _tc_body(x_ref, w0_ref, b0_ref, w1_ref, b1_ref,
             w2_ref, b2_ref, wn_ref, bn_ref, o_ref):
  args = (w0_ref, b0_ref, w1_ref, b1_ref, w2_ref, b2_ref, wn_ref, bn_ref)
  o_ref[...] = _mlp_head(x_ref[...], *args)


def _full(shape):
  return pl.BlockSpec(shape, lambda i: tuple(0 for _ in shape))


def _make_tc_dense():
  return pl.pallas_call(
      _tc_body,
      grid=(NB,),
      in_specs=[
          pl.BlockSpec((BLK, 2 * DC), lambda i: (i, 0)),
          _full((2 * DC, 128)),
          _full((1, 128)),
          _full((128, 64)),
          _full((1, 64)),
          _full((64, D)),
          _full((1, D)),
          _full((1, DC)),
          pl.BlockSpec(memory_space=pltpu.SMEM),
      ],
      out_specs=pl.BlockSpec((BLK // 128, 128), lambda i: (i, 0)),
      out_shape=jax.ShapeDtypeStruct((B // 128, 128), jnp.float32),
      compiler_params=pltpu.CompilerParams(
          dimension_semantics=("arbitrary",)),
  )


@jax.jit
def kernel(user_id, movie_title, user_gmf, movie_gmf, user_mlp, movie_mlp,
           W0, b0, W1, b1, W2, b2, Wn, bn):
  table = jnp.concatenate(
      [user_gmf, user_mlp, movie_gmf, movie_mlp], axis=1
  ).reshape(2 * user_gmf.shape[0], DC)
  uid = user_id.astype(jnp.int32).reshape(NW * NCHUNK, CHUNK)
  mid = movie_title.astype(jnp.int32).reshape(NW * NCHUNK, CHUNK)
  rows = _make_sc_gather()(uid, mid, table)
  x = rows.reshape(-1).reshape(B, 2 * DC)
  z = jnp.zeros((D, 128), jnp.float32)
  w0full = jnp.concatenate([z, W0[:D], z, W0[D:]], axis=0)
  out = _make_tc_dense()(
      x, w0full, b0.reshape(1, 128), W1,
      b1.reshape(1, 64), W2, b2.reshape(1, D), Wn.reshape(1, DC), bn)
  return out.reshape(B)
